# baseline (speedup 1.0000x reference)
import functools

import jax
import jax.numpy as jnp
from jax import lax
from jax.experimental import pallas as pl
from jax.experimental.pallas import tpu as pltpu
from jax.experimental.pallas import tpu_sc as plsc

N = 10000
E = 5000
NNZ = 160000
IN = 256
HID = 512
OUT = 256

NP_ = 10240
EP_ = 5120
NSUB = 16
NCORE = 2

f32 = jnp.float32
i32 = jnp.int32

NNZP = 163840
CK = 128
CIT = (NNZP // 32) // CK
CK2 = 80
CIT2 = (NNZP // 32) // CK2


def _mesh():
    return plsc.VectorSubcoreMesh(core_axis_name="c", subcore_axis_name="s")


def _count_edges(inc_edge, ones_rows, zeros128):
    @functools.partial(
        pl.kernel,
        out_type=jax.ShapeDtypeStruct((NCORE, EP_, 128), f32),
        mesh=_mesh(),
        scratch_types=[
            pltpu.VMEM((NNZP // 32,), i32),
            pltpu.VMEM((CK,), i32),
            pltpu.VMEM((CK,), i32),
            pltpu.VMEM((CK, 128), f32),
            pltpu.VMEM_SHARED((EP_, 128), f32),
            pltpu.SemaphoreType.DMA,
            pltpu.SemaphoreType.DMA,
        ],
    )
    def k(ie_h, ones_h, z_h, out_h, iall, idx_a, idx_b, ones_v, acc_sh,
          sem_a, sem_b):
        c = lax.axis_index("c")
        s = lax.axis_index("s")
        rows = EP_ // NSUB
        pltpu.sync_copy(z_h.at[pl.ds(0, rows)], acc_sh.at[pl.ds(s * rows, rows)])
        pltpu.sync_copy(ones_h, ones_v)
        base = (c * NSUB + s) * (NNZP // 32)
        pltpu.sync_copy(ie_h.at[pl.ds(base, NNZP // 32)], iall)
        plsc.subcore_barrier()

        def prep(i, idx_v):
            @pl.loop(0, CK // 16)
            def _(q):
                idx_v[pl.ds(q * 16, 16)] = iall[pl.ds(i * CK + q * 16, 16)]

        def sstart(idx_v, sem):
            pltpu.async_copy(ones_v, acc_sh.at[idx_v], sem, add=True)

        def swait(idx_v, sem):
            pltpu.make_async_copy(ones_v, acc_sh.at[idx_v], sem).wait()

        prep(0, idx_a)
        sstart(idx_a, sem_a)
        prep(1, idx_b)
        sstart(idx_b, sem_b)

        @pl.loop(0, (CIT - 2) // 2)
        def _(p):
            i = 2 * p
            swait(idx_a, sem_a)
            prep(i + 2, idx_a)
            sstart(idx_a, sem_a)
            swait(idx_b, sem_b)
            prep(i + 3, idx_b)
            sstart(idx_b, sem_b)

        swait(idx_a, sem_a)
        swait(idx_b, sem_b)
        plsc.subcore_barrier()
        pltpu.sync_copy(acc_sh.at[pl.ds(s * rows, rows)],
                        out_h.at[c, pl.ds(s * rows, rows)])

    return k(inc_edge, ones_rows, zeros128)


def _count_nodes(etab, inc_node, inc_edge, zeros128):
    @functools.partial(
        pl.kernel,
        out_type=jax.ShapeDtypeStruct((NCORE, NP_, 128), f32),
        mesh=_mesh(),
        scratch_types=[
            pltpu.VMEM((NNZP // 32,), i32),
            pltpu.VMEM((NNZP // 32,), i32),
            pltpu.VMEM((CK2,), i32),
            pltpu.VMEM((CK2,), i32),
            pltpu.VMEM((CK2,), i32),
            pltpu.VMEM((CK2,), i32),
            pltpu.VMEM((CK2, 128), f32),
            pltpu.VMEM((CK2, 128), f32),
            pltpu.VMEM_SHARED((NP_, 128), f32),
            pltpu.SemaphoreType.DMA,
            pltpu.SemaphoreType.DMA,
            pltpu.SemaphoreType.DMA,
            pltpu.SemaphoreType.DMA,
        ],
    )
    def k(etab_h, in_h, ie_h, z_h, out_h, eall, nall,
          eidx_a, nidx_a, eidx_b, nidx_b, rows_a, rows_b, acc_sh,
          gsem_a, ssem_a, gsem_b, ssem_b):
        c = lax.axis_index("c")
        s = lax.axis_index("s")
        rows = NP_ // NSUB
        pltpu.sync_copy(z_h.at[pl.ds(0, rows)], acc_sh.at[pl.ds(s * rows, rows)])
        base = (c * NSUB + s) * (NNZP // 32)
        pltpu.sync_copy(ie_h.at[pl.ds(base, NNZP // 32)], eall)
        pltpu.sync_copy(in_h.at[pl.ds(base, NNZP // 32)], nall)
        plsc.subcore_barrier()

        slot_a = (eidx_a, nidx_a, rows_a, gsem_a, ssem_a)
        slot_b = (eidx_b, nidx_b, rows_b, gsem_b, ssem_b)

        def prep_g(i, sl):
            eidx, nidx, rows_v, gsem, _ = sl

            @pl.loop(0, CK2 // 16)
            def _(q):
                eidx[pl.ds(q * 16, 16)] = eall[pl.ds(i * CK2 + q * 16, 16)]
                nidx[pl.ds(q * 16, 16)] = nall[pl.ds(i * CK2 + q * 16, 16)]

            pltpu.async_copy(etab_h.at[eidx], rows_v, gsem)

        def fin(sl):
            eidx, nidx, rows_v, gsem, ssem = sl
            pltpu.make_async_copy(etab_h.at[eidx], rows_v, gsem).wait()
            pltpu.async_copy(rows_v, acc_sh.at[nidx], ssem, add=True)
            pltpu.make_async_copy(rows_v, acc_sh.at[nidx], ssem).wait()

        prep_g(0, slot_a)
        prep_g(1, slot_b)

        @pl.loop(0, (CIT2 - 2) // 2)
        def _(p):
            i = 2 * p
            fin(slot_a)
            prep_g(i + 2, slot_a)
            fin(slot_b)
            prep_g(i + 3, slot_b)

        fin(slot_a)
        fin(slot_b)
        plsc.subcore_barrier()
        pltpu.sync_copy(acc_sh.at[pl.ds(s * rows, rows)],
                        out_h.at[c, pl.ds(s * rows, rows)])

    return k(etab, inc_node, inc_edge, zeros128)


def _make_stage(csrc, ssrc, sdst):
    rows_sub = sdst // NSUB
    nj = csrc // NCORE
    fk = 128 if sdst == EP_ else 80
    fit = (NNZP // NSUB) // fk
    nslot = 3 if sdst == EP_ else 2

    slot_scratch = []
    for _ in range(nslot):
        slot_scratch += [pltpu.VMEM((fk,), i32), pltpu.VMEM((fk,), i32),
                         pltpu.VMEM((fk, 128), f32),
                         pltpu.SemaphoreType.DMA, pltpu.SemaphoreType.DMA]

    @functools.partial(
        pl.kernel,
        out_type=jax.ShapeDtypeStruct((csrc, sdst, 128), f32),
        mesh=_mesh(),
        scratch_types=[
            pltpu.VMEM((NNZP // NSUB,), i32),
            pltpu.VMEM((NNZP // NSUB,), i32),
        ] + slot_scratch + [
            pltpu.VMEM_SHARED((sdst, 128), f32),
        ],
    )
    def k(src_h, gidx_h, sidx_h, z_h, out_h, gall, sall, *refs):
        slots = [tuple(refs[5 * t:5 * t + 5]) for t in range(nslot)]
        acc_sh = refs[5 * nslot]
        c = lax.axis_index("c")
        s = lax.axis_index("s")
        base = s * (NNZP // NSUB)
        pltpu.sync_copy(gidx_h.at[pl.ds(base, NNZP // NSUB)], gall)
        pltpu.sync_copy(sidx_h.at[pl.ds(base, NNZP // NSUB)], sall)

        for j in range(nj):
            chunk = c + NCORE * j
            off = chunk * ssrc
            pltpu.sync_copy(z_h.at[pl.ds(0, rows_sub)],
                            acc_sh.at[pl.ds(s * rows_sub, rows_sub)])
            plsc.subcore_barrier()

            def prep_g(i, sl, off=off):
                gidx, sidx, rows_v, gsem, _ = sl

                @pl.loop(0, fk // 16)
                def _(q):
                    gidx[pl.ds(q * 16, 16)] = (
                        gall[pl.ds(i * fk + q * 16, 16)] + off)
                    sidx[pl.ds(q * 16, 16)] = sall[pl.ds(i * fk + q * 16, 16)]

                pltpu.async_copy(src_h.at[gidx], rows_v, gsem)

            def fin(sl):
                gidx, sidx, rows_v, gsem, ssem = sl
                pltpu.make_async_copy(src_h.at[gidx], rows_v, gsem).wait()
                pltpu.async_copy(rows_v, acc_sh.at[sidx], ssem, add=True)
                pltpu.make_async_copy(rows_v, acc_sh.at[sidx], ssem).wait()

            prep_g(0, slots[0])
            prep_g(1, slots[1])

            if nslot == 2:
                @pl.loop(0, (fit - 2) // 2)
                def _(p):
                    i = 2 * p
                    fin(slots[0])
                    prep_g(i + 2, slots[0])
                    fin(slots[1])
                    prep_g(i + 3, slots[1])
            else:
                @pl.loop(0, (fit - 2) // 3)
                def _(p):
                    i = 3 * p
                    fin(slots[0])
                    prep_g(i + 2, slots[2])
                    fin(slots[1])
                    prep_g(i + 3, slots[0])
                    fin(slots[2])
                    prep_g(i + 4, slots[1])

            fin(slots[0])
            fin(slots[1])
            plsc.subcore_barrier()
            pltpu.sync_copy(acc_sh.at[pl.ds(s * rows_sub, rows_sub)],
                            out_h.at[chunk, pl.ds(s * rows_sub, rows_sub)])

    return k


_stage_e2 = _make_stage(2, NP_, EP_)
_stage_n2 = _make_stage(2, EP_, NP_)


def _edge_tables(ecnt):
    def body(cnt_ref, etab_ref, escale_ref):
        cnt = cnt_ref[0] + cnt_ref[1]
        esize = jnp.maximum(cnt[:, 0:1], 1.0)
        lane = lax.broadcasted_iota(i32, (EP_, 128), 1)
        etab_ref[...] = jnp.where(lane == 0, esize,
                                  jnp.where(lane == 1, 1.0, 0.0))
        escale_ref[...] = jnp.broadcast_to(1.0 / esize, (EP_, 16))

    return pl.pallas_call(
        body,
        out_shape=[jax.ShapeDtypeStruct((EP_, 128), f32),
                   jax.ShapeDtypeStruct((EP_, 16), f32)],
    )(ecnt)


def _node_scale(nacc):
    def body(a_ref, out_ref):
        p = a_ref[0, :, 0:16] + a_ref[1, :, 0:16]
        nnb = jnp.maximum(p[:, 0:1], 1.0)
        deg = jnp.maximum(p[:, 1:2], 1.0)
        out_ref[...] = jnp.broadcast_to(1.0 / (deg * nnb), (NP_, 16))

    return pl.pallas_call(
        body, out_shape=jax.ShapeDtypeStruct((NP_, 16), f32))(nacc)


def _edge_norm(eacc, escale, csrc):
    def body(a_ref, s_ref, o_ref):
        o_ref[...] = a_ref[...] * s_ref[:, 0:1]

    return pl.pallas_call(
        body,
        grid=(csrc,),
        in_specs=[pl.BlockSpec((EP_, 128), lambda c: (c, 0)),
                  pl.BlockSpec((EP_, 16), lambda c: (0, 0))],
        out_specs=pl.BlockSpec((EP_, 128), lambda c: (c, 0)),
        out_shape=jax.ShapeDtypeStruct((csrc * EP_, 128), f32),
    )(eacc.reshape(csrc * EP_, 128), escale)


def _mm_top(xch, w, csrc, h):
    BN, BH = 512, 256
    tn, th = NP_ // BN, h // BH

    def body(x_ref, w_ref, o_ref, acc):
        kc = pl.program_id(2)

        @pl.when(kc == 0)
        def _():
            acc[...] = jnp.zeros_like(acc)

        acc[...] += jnp.dot(x_ref[...].astype(jnp.bfloat16),
                            w_ref[...].astype(jnp.bfloat16),
                            preferred_element_type=f32)

        @pl.when(kc == csrc - 1)
        def _():
            o_ref[...] = acc[...]

    return pl.pallas_call(
        body,
        grid=(tn, th, csrc),
        in_specs=[pl.BlockSpec((BN, 128), lambda i, j, k: (k * tn + i, 0)),
                  pl.BlockSpec((128, BH), lambda i, j, k: (k, j))],
        out_specs=pl.BlockSpec((BN, BH), lambda i, j, k: (i, j)),
        out_shape=jax.ShapeDtypeStruct((NP_, h), f32),
        scratch_shapes=[pltpu.VMEM((BN, BH), f32)],
        compiler_params=pltpu.CompilerParams(
            dimension_semantics=("parallel", "parallel", "arbitrary")),
    )(xch, w)


def _mm_bot(htop, nacc, nscale, w, csrc, h, chunk_out, act=True):
    BN, BH = 1024, 128 if chunk_out else 256
    tn, th = NP_ // BN, h // BH

    def body(t_ref, a_ref, sc_ref, w_ref, o_ref, acc):
        kc = pl.program_id(2)

        @pl.when(kc == 0)
        def _():
            acc[...] = t_ref[...]

        agg = a_ref[...] * sc_ref[:, 0:1]
        acc[...] += jnp.dot(agg.astype(jnp.bfloat16),
                            w_ref[...].astype(jnp.bfloat16),
                            preferred_element_type=f32)

        @pl.when(kc == csrc - 1)
        def _():
            v = acc[...]
            o_ref[...] = jnp.where(v >= 0, v, f32(0.01) * v) if act else v

    if chunk_out:
        out_shape = jax.ShapeDtypeStruct((th * NP_, 128), f32)
        out_spec = pl.BlockSpec((BN, 128), lambda i, j, k: (j * tn + i, 0))
    else:
        out_shape = jax.ShapeDtypeStruct((NP_, h), f32)
        out_spec = pl.BlockSpec((BN, BH), lambda i, j, k: (i, j))

    return pl.pallas_call(
        body,
        grid=(tn, th, csrc),
        in_specs=[pl.BlockSpec((BN, BH), lambda i, j, k: (i, j)),
                  pl.BlockSpec((BN, 128), lambda i, j, k: (k * tn + i, 0)),
                  pl.BlockSpec((BN, 16), lambda i, j, k: (i, 0)),
                  pl.BlockSpec((128, BH), lambda i, j, k: (k, j))],
        out_specs=out_spec,
        out_shape=out_shape,
        scratch_shapes=[pltpu.VMEM((BN, BH), f32)],
        compiler_params=pltpu.CompilerParams(
            dimension_semantics=("parallel", "parallel", "arbitrary")),
    )(htop, nacc, nscale, w)


def kernel(node_feat, inc_node, inc_edge, W1, W2):
    inc_node = inc_node.astype(i32)
    inc_edge = inc_edge.astype(i32)

    x_p = jnp.pad(node_feat, ((0, NP_ - N), (0, 0)))
    xch = x_p.reshape(NP_, 2, 128).transpose(1, 0, 2).reshape(2 * NP_, 128)

    zeros128 = jnp.zeros((NP_, 128), f32)
    ones_rows = jnp.zeros((CK, 128), f32).at[:, 0].set(1.0)

    pad = NNZP - NNZ
    ie_cnt = jnp.concatenate(
        [inc_edge, E + (jnp.arange(pad, dtype=i32) % (EP_ - E))])
    in_cnt = jnp.concatenate(
        [inc_node, N + (jnp.arange(pad, dtype=i32) % (NP_ - N))])

    ecnt = _count_edges(ie_cnt, ones_rows, zeros128)
    etab, escale = _edge_tables(ecnt)
    ncnt = _count_nodes(etab, in_cnt, ie_cnt, zeros128)
    nscale = _node_scale(ncnt)

    eacc1 = _stage_e2(xch, in_cnt, ie_cnt, zeros128)
    eemb1 = _edge_norm(eacc1, escale, 2)
    nacc1 = _stage_n2(eemb1, ie_cnt, in_cnt, zeros128)
    htop1 = _mm_top(xch, W1[:IN], 2, HID)
    nacc1r = nacc1.reshape(2 * NP_, 128)
    hh = HID // 2
    x2a = _mm_bot(htop1[:, :hh], nacc1r, nscale, W1[IN:, :hh], 2, hh,
                  chunk_out=True)
    x2b = _mm_bot(htop1[:, hh:], nacc1r, nscale, W1[IN:, hh:], 2, hh,
                  chunk_out=True)

    eacc2a = _stage_e2(x2a, in_cnt, ie_cnt, zeros128)
    eemb2a = _edge_norm(eacc2a, escale, 2)
    nacc2a = _stage_n2(eemb2a, ie_cnt, in_cnt, zeros128)
    eacc2b = _stage_e2(x2b, in_cnt, ie_cnt, zeros128)
    eemb2b = _edge_norm(eacc2b, escale, 2)
    nacc2b = _stage_n2(eemb2b, ie_cnt, in_cnt, zeros128)
    htop2 = (_mm_top(x2a, W2[:hh], 2, OUT)
             + _mm_top(x2b, W2[hh:HID], 2, OUT))
    bot2a = _mm_bot(htop2, nacc2a.reshape(2 * NP_, 128), nscale,
                    W2[HID:HID + hh], 2, OUT, chunk_out=False, act=False)
    out = _mm_bot(bot2a, nacc2b.reshape(2 * NP_, 128), nscale,
                  W2[HID + hh:], 2, OUT, chunk_out=False, act=True)

    return out[:N]


# speedup vs baseline: 1.0345x; 1.0345x over previous
import functools

import jax
import jax.numpy as jnp
from jax import lax
from jax.experimental import pallas as pl
from jax.experimental.pallas import tpu as pltpu
from jax.experimental.pallas import tpu_sc as plsc

N = 10000
E = 5000
NNZ = 160000
IN = 256
HID = 512
OUT = 256

NP_ = 10240
EP_ = 5120
NSUB = 16
NCORE = 2

f32 = jnp.float32
i32 = jnp.int32

NNZP = 163840
CK = 128
CIT = (NNZP // 32) // CK
CK2 = 80
CIT2 = (NNZP // 32) // CK2


def _mesh():
    return plsc.VectorSubcoreMesh(core_axis_name="c", subcore_axis_name="s")


def _count_edges(inc_edge, ones_rows, zeros128):
    @functools.partial(
        pl.kernel,
        out_type=jax.ShapeDtypeStruct((NCORE, EP_, 128), f32),
        mesh=_mesh(),
        scratch_types=[
            pltpu.VMEM((NNZP // 32,), i32),
            pltpu.VMEM((CK,), i32),
            pltpu.VMEM((CK,), i32),
            pltpu.VMEM((CK, 128), f32),
            pltpu.VMEM_SHARED((EP_, 128), f32),
            pltpu.SemaphoreType.DMA,
            pltpu.SemaphoreType.DMA,
        ],
    )
    def k(ie_h, ones_h, z_h, out_h, iall, idx_a, idx_b, ones_v, acc_sh,
          sem_a, sem_b):
        c = lax.axis_index("c")
        s = lax.axis_index("s")
        rows = EP_ // NSUB
        pltpu.sync_copy(z_h.at[pl.ds(0, rows)], acc_sh.at[pl.ds(s * rows, rows)])
        pltpu.sync_copy(ones_h, ones_v)
        base = (c * NSUB + s) * (NNZP // 32)
        pltpu.sync_copy(ie_h.at[pl.ds(base, NNZP // 32)], iall)
        plsc.subcore_barrier()

        def prep(i, idx_v):
            @pl.loop(0, CK // 16)
            def _(q):
                idx_v[pl.ds(q * 16, 16)] = iall[pl.ds(i * CK + q * 16, 16)]

        def sstart(idx_v, sem):
            pltpu.async_copy(ones_v, acc_sh.at[idx_v], sem, add=True)

        def swait(idx_v, sem):
            pltpu.make_async_copy(ones_v, acc_sh.at[idx_v], sem).wait()

        prep(0, idx_a)
        sstart(idx_a, sem_a)
        prep(1, idx_b)
        sstart(idx_b, sem_b)

        @pl.loop(0, (CIT - 2) // 2)
        def _(p):
            i = 2 * p
            swait(idx_a, sem_a)
            prep(i + 2, idx_a)
            sstart(idx_a, sem_a)
            swait(idx_b, sem_b)
            prep(i + 3, idx_b)
            sstart(idx_b, sem_b)

        swait(idx_a, sem_a)
        swait(idx_b, sem_b)
        plsc.subcore_barrier()
        pltpu.sync_copy(acc_sh.at[pl.ds(s * rows, rows)],
                        out_h.at[c, pl.ds(s * rows, rows)])

    return k(inc_edge, ones_rows, zeros128)


def _count_nodes(etab, inc_node, inc_edge, zeros128):
    @functools.partial(
        pl.kernel,
        out_type=jax.ShapeDtypeStruct((NCORE, NP_, 128), f32),
        mesh=_mesh(),
        scratch_types=[
            pltpu.VMEM((NNZP // 32,), i32),
            pltpu.VMEM((NNZP // 32,), i32),
            pltpu.VMEM((CK2,), i32),
            pltpu.VMEM((CK2,), i32),
            pltpu.VMEM((CK2,), i32),
            pltpu.VMEM((CK2,), i32),
            pltpu.VMEM((CK2, 128), f32),
            pltpu.VMEM((CK2, 128), f32),
            pltpu.VMEM_SHARED((NP_, 128), f32),
            pltpu.SemaphoreType.DMA,
            pltpu.SemaphoreType.DMA,
            pltpu.SemaphoreType.DMA,
            pltpu.SemaphoreType.DMA,
        ],
    )
    def k(etab_h, in_h, ie_h, z_h, out_h, eall, nall,
          eidx_a, nidx_a, eidx_b, nidx_b, rows_a, rows_b, acc_sh,
          gsem_a, ssem_a, gsem_b, ssem_b):
        c = lax.axis_index("c")
        s = lax.axis_index("s")
        rows = NP_ // NSUB
        pltpu.sync_copy(z_h.at[pl.ds(0, rows)], acc_sh.at[pl.ds(s * rows, rows)])
        base = (c * NSUB + s) * (NNZP // 32)
        pltpu.sync_copy(ie_h.at[pl.ds(base, NNZP // 32)], eall)
        pltpu.sync_copy(in_h.at[pl.ds(base, NNZP // 32)], nall)
        plsc.subcore_barrier()

        slot_a = (eidx_a, nidx_a, rows_a, gsem_a, ssem_a)
        slot_b = (eidx_b, nidx_b, rows_b, gsem_b, ssem_b)

        def prep_g(i, sl):
            eidx, nidx, rows_v, gsem, _ = sl

            @pl.loop(0, CK2 // 16)
            def _(q):
                eidx[pl.ds(q * 16, 16)] = eall[pl.ds(i * CK2 + q * 16, 16)]
                nidx[pl.ds(q * 16, 16)] = nall[pl.ds(i * CK2 + q * 16, 16)]

            pltpu.async_copy(etab_h.at[eidx], rows_v, gsem)

        def fin(sl):
            eidx, nidx, rows_v, gsem, ssem = sl
            pltpu.make_async_copy(etab_h.at[eidx], rows_v, gsem).wait()
            pltpu.async_copy(rows_v, acc_sh.at[nidx], ssem, add=True)
            pltpu.make_async_copy(rows_v, acc_sh.at[nidx], ssem).wait()

        prep_g(0, slot_a)
        prep_g(1, slot_b)

        @pl.loop(0, (CIT2 - 2) // 2)
        def _(p):
            i = 2 * p
            fin(slot_a)
            prep_g(i + 2, slot_a)
            fin(slot_b)
            prep_g(i + 3, slot_b)

        fin(slot_a)
        fin(slot_b)
        plsc.subcore_barrier()
        pltpu.sync_copy(acc_sh.at[pl.ds(s * rows, rows)],
                        out_h.at[c, pl.ds(s * rows, rows)])

    return k(etab, inc_node, inc_edge, zeros128)


def _make_stage(csrc, ssrc, sdst):
    rows_sub = sdst // NSUB
    nj = csrc // NCORE
    fk = 128 if sdst == EP_ else 80
    fit = (NNZP // NSUB) // fk
    nslot = 3 if sdst == EP_ else 2

    slot_scratch = []
    for _ in range(nslot):
        slot_scratch += [pltpu.VMEM((fk,), i32), pltpu.VMEM((fk,), i32),
                         pltpu.VMEM((fk, 128), f32),
                         pltpu.SemaphoreType.DMA, pltpu.SemaphoreType.DMA]

    @functools.partial(
        pl.kernel,
        out_type=jax.ShapeDtypeStruct((csrc, sdst, 128), f32),
        mesh=_mesh(),
        scratch_types=[
            pltpu.VMEM((NNZP // NSUB,), i32),
            pltpu.VMEM((NNZP // NSUB,), i32),
        ] + slot_scratch + [
            pltpu.VMEM_SHARED((sdst, 128), f32),
        ],
    )
    def k(src_h, gidx_h, sidx_h, z_h, out_h, gall, sall, *refs):
        slots = [tuple(refs[5 * t:5 * t + 5]) for t in range(nslot)]
        acc_sh = refs[5 * nslot]
        c = lax.axis_index("c")
        s = lax.axis_index("s")
        base = s * (NNZP // NSUB)
        pltpu.sync_copy(gidx_h.at[pl.ds(base, NNZP // NSUB)], gall)
        pltpu.sync_copy(sidx_h.at[pl.ds(base, NNZP // NSUB)], sall)

        for j in range(nj):
            chunk = c + NCORE * j
            off = chunk * ssrc
            pltpu.sync_copy(z_h.at[pl.ds(0, rows_sub)],
                            acc_sh.at[pl.ds(s * rows_sub, rows_sub)])
            plsc.subcore_barrier()

            def prep_g(i, sl, off=off):
                gidx, sidx, rows_v, gsem, _ = sl

                @pl.loop(0, fk // 16)
                def _(q):
                    gidx[pl.ds(q * 16, 16)] = (
                        gall[pl.ds(i * fk + q * 16, 16)] + off)
                    sidx[pl.ds(q * 16, 16)] = sall[pl.ds(i * fk + q * 16, 16)]

                pltpu.async_copy(src_h.at[gidx], rows_v, gsem)

            def fin(sl):
                gidx, sidx, rows_v, gsem, ssem = sl
                pltpu.make_async_copy(src_h.at[gidx], rows_v, gsem).wait()
                pltpu.async_copy(rows_v, acc_sh.at[sidx], ssem, add=True)
                pltpu.make_async_copy(rows_v, acc_sh.at[sidx], ssem).wait()

            prep_g(0, slots[0])
            prep_g(1, slots[1])

            if nslot == 2:
                @pl.loop(0, (fit - 2) // 2)
                def _(p):
                    i = 2 * p
                    fin(slots[0])
                    prep_g(i + 2, slots[0])
                    fin(slots[1])
                    prep_g(i + 3, slots[1])
            else:
                @pl.loop(0, (fit - 2) // 3)
                def _(p):
                    i = 3 * p
                    fin(slots[0])
                    prep_g(i + 2, slots[2])
                    fin(slots[1])
                    prep_g(i + 3, slots[0])
                    fin(slots[2])
                    prep_g(i + 4, slots[1])

            fin(slots[0])
            fin(slots[1])
            plsc.subcore_barrier()
            pltpu.sync_copy(acc_sh.at[pl.ds(s * rows_sub, rows_sub)],
                            out_h.at[chunk, pl.ds(s * rows_sub, rows_sub)])

    return k


_stage_e2 = _make_stage(2, NP_, EP_)
_stage_n2 = _make_stage(2, EP_, NP_)
_stage_e4 = _make_stage(4, NP_, EP_)
_stage_n4 = _make_stage(4, EP_, NP_)


def _edge_tables(ecnt):
    def body(cnt_ref, etab_ref, escale_ref):
        cnt = cnt_ref[0] + cnt_ref[1]
        esize = jnp.maximum(cnt[:, 0:1], 1.0)
        lane = lax.broadcasted_iota(i32, (EP_, 128), 1)
        etab_ref[...] = jnp.where(lane == 0, esize,
                                  jnp.where(lane == 1, 1.0, 0.0))
        escale_ref[...] = jnp.broadcast_to(1.0 / esize, (EP_, 16))

    return pl.pallas_call(
        body,
        out_shape=[jax.ShapeDtypeStruct((EP_, 128), f32),
                   jax.ShapeDtypeStruct((EP_, 16), f32)],
    )(ecnt)


def _node_scale(nacc):
    def body(a_ref, out_ref):
        p = a_ref[0, :, 0:16] + a_ref[1, :, 0:16]
        nnb = jnp.maximum(p[:, 0:1], 1.0)
        deg = jnp.maximum(p[:, 1:2], 1.0)
        out_ref[...] = jnp.broadcast_to(1.0 / (deg * nnb), (NP_, 16))

    return pl.pallas_call(
        body, out_shape=jax.ShapeDtypeStruct((NP_, 16), f32))(nacc)


def _edge_norm(eacc, escale, csrc):
    def body(a_ref, s_ref, o_ref):
        o_ref[...] = a_ref[...] * s_ref[:, 0:1]

    return pl.pallas_call(
        body,
        grid=(csrc,),
        in_specs=[pl.BlockSpec((EP_, 128), lambda c: (c, 0)),
                  pl.BlockSpec((EP_, 16), lambda c: (0, 0))],
        out_specs=pl.BlockSpec((EP_, 128), lambda c: (c, 0)),
        out_shape=jax.ShapeDtypeStruct((csrc * EP_, 128), f32),
    )(eacc.reshape(csrc * EP_, 128), escale)


def _mm_top(xch, w, csrc, h):
    BN, BH = 512, 256
    tn, th = NP_ // BN, h // BH

    def body(x_ref, w_ref, o_ref, acc):
        kc = pl.program_id(2)

        @pl.when(kc == 0)
        def _():
            acc[...] = jnp.zeros_like(acc)

        acc[...] += jnp.dot(x_ref[...].astype(jnp.bfloat16),
                            w_ref[...].astype(jnp.bfloat16),
                            preferred_element_type=f32)

        @pl.when(kc == csrc - 1)
        def _():
            o_ref[...] = acc[...]

    return pl.pallas_call(
        body,
        grid=(tn, th, csrc),
        in_specs=[pl.BlockSpec((BN, 128), lambda i, j, k: (k * tn + i, 0)),
                  pl.BlockSpec((128, BH), lambda i, j, k: (k, j))],
        out_specs=pl.BlockSpec((BN, BH), lambda i, j, k: (i, j)),
        out_shape=jax.ShapeDtypeStruct((NP_, h), f32),
        scratch_shapes=[pltpu.VMEM((BN, BH), f32)],
        compiler_params=pltpu.CompilerParams(
            dimension_semantics=("parallel", "parallel", "arbitrary")),
    )(xch, w)


def _mm_bot(htop, nacc, nscale, w, csrc, h, chunk_out):
    BN, BH = 1024, 128 if chunk_out else 256
    tn, th = NP_ // BN, h // BH

    def body(t_ref, a_ref, sc_ref, w_ref, o_ref, acc):
        kc = pl.program_id(2)

        @pl.when(kc == 0)
        def _():
            acc[...] = t_ref[...]

        agg = a_ref[...] * sc_ref[:, 0:1]
        acc[...] += jnp.dot(agg.astype(jnp.bfloat16),
                            w_ref[...].astype(jnp.bfloat16),
                            preferred_element_type=f32)

        @pl.when(kc == csrc - 1)
        def _():
            v = acc[...]
            o_ref[...] = jnp.where(v >= 0, v, f32(0.01) * v)

    if chunk_out:
        out_shape = jax.ShapeDtypeStruct((th * NP_, 128), f32)
        out_spec = pl.BlockSpec((BN, 128), lambda i, j, k: (j * tn + i, 0))
    else:
        out_shape = jax.ShapeDtypeStruct((NP_, h), f32)
        out_spec = pl.BlockSpec((BN, BH), lambda i, j, k: (i, j))

    return pl.pallas_call(
        body,
        grid=(tn, th, csrc),
        in_specs=[pl.BlockSpec((BN, BH), lambda i, j, k: (i, j)),
                  pl.BlockSpec((BN, 128), lambda i, j, k: (k * tn + i, 0)),
                  pl.BlockSpec((BN, 16), lambda i, j, k: (i, 0)),
                  pl.BlockSpec((128, BH), lambda i, j, k: (k, j))],
        out_specs=out_spec,
        out_shape=out_shape,
        scratch_shapes=[pltpu.VMEM((BN, BH), f32)],
        compiler_params=pltpu.CompilerParams(
            dimension_semantics=("parallel", "parallel", "arbitrary")),
    )(htop, nacc, nscale, w)


def kernel(node_feat, inc_node, inc_edge, W1, W2):
    inc_node = inc_node.astype(i32)
    inc_edge = inc_edge.astype(i32)

    x_p = jnp.pad(node_feat, ((0, NP_ - N), (0, 0)))
    xch = x_p.reshape(NP_, 2, 128).transpose(1, 0, 2).reshape(2 * NP_, 128)

    zeros128 = jnp.zeros((NP_, 128), f32)
    ones_rows = jnp.zeros((CK, 128), f32).at[:, 0].set(1.0)

    pad = NNZP - NNZ
    ie_cnt = jnp.concatenate(
        [inc_edge, E + (jnp.arange(pad, dtype=i32) % (EP_ - E))])
    in_cnt = jnp.concatenate(
        [inc_node, N + (jnp.arange(pad, dtype=i32) % (NP_ - N))])

    ecnt = _count_edges(ie_cnt, ones_rows, zeros128)
    etab, escale = _edge_tables(ecnt)
    ncnt = _count_nodes(etab, in_cnt, ie_cnt, zeros128)
    nscale = _node_scale(ncnt)

    eacc1 = _stage_e2(xch, in_cnt, ie_cnt, zeros128)
    eemb1 = _edge_norm(eacc1, escale, 2)
    nacc1 = _stage_n2(eemb1, ie_cnt, in_cnt, zeros128)
    htop1 = _mm_top(xch, W1[:IN], 2, HID)
    x2ch = _mm_bot(htop1, nacc1.reshape(2 * NP_, 128), nscale,
                   W1[IN:], 2, HID, chunk_out=True)

    eacc2 = _stage_e4(x2ch, in_cnt, ie_cnt, zeros128)
    eemb2 = _edge_norm(eacc2, escale, 4)
    nacc2 = _stage_n4(eemb2, ie_cnt, in_cnt, zeros128)
    htop2 = _mm_top(x2ch, W2[:HID], 4, OUT)
    out = _mm_bot(htop2, nacc2.reshape(4 * NP_, 128), nscale,
                  W2[HID:], 4, OUT, chunk_out=False)

    return out[:N]
